# baseline (speedup 1.0000x reference)
import jax
import jax.numpy as jnp
from jax.experimental import pallas as pl
from jax.experimental.pallas import tpu as pltpu

B = 16384
D = 64
SEM_T = 0.3
IL_COEF = 0.7
BL_COEF = 0.1
RZ_COEF = 0.01

BLOCK = 1024
HALF = BLOCK // 2


def _silu(x):
    return x * (0.5 + 0.5 * jnp.tanh(0.5 * x))


def _dgen(a, b, ca, cb):
    return jax.lax.dot_general(a, b, (((ca,), (cb,)), ((), ())),
                               preferred_element_type=jnp.float32)


def _iota2(shape, dim):
    return jax.lax.broadcasted_iota(jnp.int32, shape, dim)


def _pair(v):
    return jnp.concatenate([v[:HALF, :], v[HALF:, :]], axis=1)


def _fused_kernel(et_ref, ei_ref, mt_ref, mi_ref,
                  aW1_ref, ab1_ref, aW2_ref, ab2_ref,
                  gW1_ref, gb1_ref, gW2_ref, gb2_ref,
                  mask_ref, loss_ref, acc_ref):
    i = pl.program_id(0)
    nblk = pl.num_programs(0)
    f32 = jnp.float32

    @pl.when(i == 0)
    def _init():
        acc_ref[0] = 0.0
        acc_ref[1] = 0.0
        acc_ref[2] = 0.0

    x_et = _pair(et_ref[...])
    x_ei = _pair(ei_ref[...])
    x_mt = _pair(mt_ref[...])
    x_mi = _pair(mi_ref[...])

    dmask = (_iota2((2 * D, 2 * D), 0) // D) == (_iota2((2 * D, 2 * D), 1) // D)
    aW1d = jnp.where(dmask, jnp.tile(aW1_ref[...], (2, 2)), 0.0)
    gW1d = jnp.where(dmask, jnp.tile(gW1_ref[...], (2, 2)), 0.0)
    ab1d = jnp.tile(ab1_ref[...], (1, 2))
    gb1d = jnp.tile(gb1_ref[...], (1, 2))
    segsel = ((_iota2((2, 2 * D), 1) // D) == _iota2((2, 2 * D), 0)).astype(f32)
    w2pT = segsel * jnp.tile(jnp.transpose(aW2_ref[...]), (1, 2))
    gw2pT = jnp.where(
        (_iota2((4, 2 * D), 1) // D) == (_iota2((4, 2 * D), 0) // 2),
        jnp.tile(jnp.transpose(gW2_ref[...]), (2, 2)), 0.0)
    gb2T = jnp.transpose(jnp.tile(gb2_ref[...], (1, 2)))

    def score_t(xp):
        h = _silu(_dgen(xp, aW1d, 1, 0) + ab1d)
        return _dgen(w2pT, h, 1, 1) + ab2_ref[0, 0]

    s_et = score_t(x_et)
    s_ei = score_t(x_ei)
    s_mt = score_t(x_mt)
    s_mi = score_t(x_mi)

    gate_in = (x_et * _dgen(s_et, segsel, 0, 0)
               + x_ei * _dgen(s_ei, segsel, 0, 0)
               + x_mt * _dgen(s_mt, segsel, 0, 0)
               + x_mi * _dgen(s_mi, segsel, 0, 0))

    g = _silu(_dgen(gate_in, gW1d, 1, 0) + gb1d)
    logitsT = _dgen(gw2pT, g, 1, 1) + gb2T

    dotT = _dgen(segsel, x_mt * x_mi, 1, 1)
    ntT = _dgen(segsel, x_mt * x_mt, 1, 1)
    niT = _dgen(segsel, x_mi * x_mi, 1, 1)
    clip = dotT * jax.lax.rsqrt(ntT) * jax.lax.rsqrt(niT)
    sem1 = clip > SEM_T

    l0 = jnp.concatenate([logitsT[0:1, :], logitsT[2:3, :]], axis=0)
    l1 = jnp.concatenate([logitsT[1:2, :], logitsT[3:4, :]], axis=0)
    d = l0 - l1
    t = jnp.log1p(jnp.exp(-jnp.abs(d)))
    picked = -(t + jnp.where(sem1, jnp.maximum(d, 0.0), jnp.maximum(-d, 0.0)))
    lse = jnp.maximum(l0, l1) + t
    acc_ref[0] += jnp.sum(picked)
    acc_ref[1] += jnp.sum(lse * lse)
    acc_ref[2] += jnp.sum((l1 > l0).astype(f32))

    p0 = 0.5 + 0.5 * jnp.tanh(0.5 * d)
    p1 = 1.0 - p0
    eye4 = (_iota2((4, 4), 0) == _iota2((4, 4), 1)).astype(f32)

    def mask_rows(k):
        mT = jnp.concatenate([p0[k:k + 1], p0[k:k + 1],
                              p1[k:k + 1], p1[k:k + 1]], axis=0)
        return _dgen(mT, eye4, 0, 0)

    mask_ref[0:HALF, :] = mask_rows(0)
    mask_ref[HALF:BLOCK, :] = mask_rows(1)

    @pl.when(i == nblk - 1)
    def _final():
        inv_b = 1.0 / B
        interaction = IL_COEF * (-(acc_ref[0] * inv_b))
        router_z = RZ_COEF * (RZ_COEF * (acc_ref[1] * inv_b))
        d1 = acc_ref[2] * inv_b
        balance = BL_COEF * (d1 - 0.5) * (d1 - 0.5)
        loss_ref[0, 0] = interaction + router_z + balance


@jax.jit
def _run(e_t, e_i, m_t, m_i, attn_W1, attn_b1, attn_W2, attn_b2,
         gate_W1, gate_b1, gate_W2, gate_b2):
    nblk = B // BLOCK
    row_spec = pl.BlockSpec((BLOCK, D), lambda i: (i, 0))
    full = lambda shape: pl.BlockSpec(shape, lambda i: (0,) * len(shape))

    mask, loss = pl.pallas_call(
        _fused_kernel,
        grid=(nblk,),
        in_specs=[
            row_spec, row_spec, row_spec, row_spec,
            full((D, D)), full((1, D)), full((D, 1)), full((1, 1)),
            full((D, D)), full((1, D)), full((D, 2)), full((1, 2)),
        ],
        out_specs=[
            pl.BlockSpec((BLOCK, 4), lambda i: (i, 0)),
            pl.BlockSpec(memory_space=pltpu.SMEM),
        ],
        out_shape=[
            jax.ShapeDtypeStruct((B, 4), jnp.float32),
            jax.ShapeDtypeStruct((1, 1), jnp.float32),
        ],
        scratch_shapes=[pltpu.SMEM((3,), jnp.float32)],
    )(e_t, e_i, m_t, m_i,
      attn_W1, attn_b1.reshape(1, D), attn_W2, attn_b2.reshape(1, 1),
      gate_W1, gate_b1.reshape(1, D), gate_W2, gate_b2.reshape(1, 2))
    return mask, loss[0, 0]


def kernel(p_t, p_i, e_t, e_i, m_t, m_i, attn_W1, attn_b1, attn_W2, attn_b2,
           gate_W1, gate_b1, gate_W2, gate_b2):
    return _run(e_t, e_i, m_t, m_i, attn_W1, attn_b1, attn_W2, attn_b2,
                gate_W1, gate_b1, gate_W2, gate_b2)


# speedup vs baseline: 1.1305x; 1.1305x over previous
import jax
import jax.numpy as jnp
from jax.experimental import pallas as pl
from jax.experimental.pallas import tpu as pltpu

B = 16384
D = 64
SEM_T = 0.3
IL_COEF = 0.7
BL_COEF = 0.1
RZ_COEF = 0.01

BLOCK = 8192
HALF = BLOCK // 2


def _silu(x):
    return x * (0.5 + 0.5 * jnp.tanh(0.5 * x))


def _dgen(a, b, ca, cb):
    return jax.lax.dot_general(a, b, (((ca,), (cb,)), ((), ())),
                               preferred_element_type=jnp.float32)


def _iota2(shape, dim):
    return jax.lax.broadcasted_iota(jnp.int32, shape, dim)


def _pair(v):
    return jnp.concatenate([v[:HALF, :], v[HALF:, :]], axis=1)


def _fused_kernel(et_ref, ei_ref, mt_ref, mi_ref,
                  aW1_ref, ab1_ref, aW2_ref, ab2_ref,
                  gW1_ref, gb1_ref, gW2_ref, gb2_ref,
                  mask_ref, loss_ref, acc_ref):
    i = pl.program_id(0)
    nblk = pl.num_programs(0)
    f32 = jnp.float32

    @pl.when(i == 0)
    def _init():
        acc_ref[0] = 0.0
        acc_ref[1] = 0.0
        acc_ref[2] = 0.0

    x_et = _pair(et_ref[...])
    x_ei = _pair(ei_ref[...])
    x_mt = _pair(mt_ref[...])
    x_mi = _pair(mi_ref[...])

    dmask = (_iota2((2 * D, 2 * D), 0) // D) == (_iota2((2 * D, 2 * D), 1) // D)
    aW1d = jnp.where(dmask, jnp.tile(aW1_ref[...], (2, 2)), 0.0)
    gW1d = jnp.where(dmask, jnp.tile(gW1_ref[...], (2, 2)), 0.0)
    ab1d = jnp.tile(ab1_ref[...], (1, 2))
    gb1d = jnp.tile(gb1_ref[...], (1, 2))
    segsel = ((_iota2((2, 2 * D), 1) // D) == _iota2((2, 2 * D), 0)).astype(f32)
    w2pT = segsel * jnp.tile(jnp.transpose(aW2_ref[...]), (1, 2))
    gw2pT = jnp.where(
        (_iota2((4, 2 * D), 1) // D) == (_iota2((4, 2 * D), 0) // 2),
        jnp.tile(jnp.transpose(gW2_ref[...]), (2, 2)), 0.0)
    gb2T = jnp.transpose(jnp.tile(gb2_ref[...], (1, 2)))

    def score_t(xp):
        h = _silu(_dgen(xp, aW1d, 1, 0) + ab1d)
        return _dgen(w2pT, h, 1, 1) + ab2_ref[0, 0]

    s_et = score_t(x_et)
    s_ei = score_t(x_ei)
    s_mt = score_t(x_mt)
    s_mi = score_t(x_mi)

    gate_in = (x_et * _dgen(s_et, segsel, 0, 0)
               + x_ei * _dgen(s_ei, segsel, 0, 0)
               + x_mt * _dgen(s_mt, segsel, 0, 0)
               + x_mi * _dgen(s_mi, segsel, 0, 0))

    g = _silu(_dgen(gate_in, gW1d, 1, 0) + gb1d)
    logitsT = _dgen(gw2pT, g, 1, 1) + gb2T

    dotT = _dgen(segsel, x_mt * x_mi, 1, 1)
    ntT = _dgen(segsel, x_mt * x_mt, 1, 1)
    niT = _dgen(segsel, x_mi * x_mi, 1, 1)
    clip = dotT * jax.lax.rsqrt(ntT) * jax.lax.rsqrt(niT)
    sem1 = clip > SEM_T

    l0 = jnp.concatenate([logitsT[0:1, :], logitsT[2:3, :]], axis=0)
    l1 = jnp.concatenate([logitsT[1:2, :], logitsT[3:4, :]], axis=0)
    d = l0 - l1
    t = jnp.log1p(jnp.exp(-jnp.abs(d)))
    picked = -(t + jnp.where(sem1, jnp.maximum(d, 0.0), jnp.maximum(-d, 0.0)))
    lse = jnp.maximum(l0, l1) + t
    acc_ref[0] += jnp.sum(picked)
    acc_ref[1] += jnp.sum(lse * lse)
    acc_ref[2] += jnp.sum((l1 > l0).astype(f32))

    p0 = 0.5 + 0.5 * jnp.tanh(0.5 * d)
    p1 = 1.0 - p0
    eye4 = (_iota2((4, 4), 0) == _iota2((4, 4), 1)).astype(f32)

    def mask_rows(k):
        mT = jnp.concatenate([p0[k:k + 1], p0[k:k + 1],
                              p1[k:k + 1], p1[k:k + 1]], axis=0)
        return _dgen(mT, eye4, 0, 0)

    mask_ref[0:HALF, :] = mask_rows(0)
    mask_ref[HALF:BLOCK, :] = mask_rows(1)

    @pl.when(i == nblk - 1)
    def _final():
        inv_b = 1.0 / B
        interaction = IL_COEF * (-(acc_ref[0] * inv_b))
        router_z = RZ_COEF * (RZ_COEF * (acc_ref[1] * inv_b))
        d1 = acc_ref[2] * inv_b
        balance = BL_COEF * (d1 - 0.5) * (d1 - 0.5)
        loss_ref[0, 0] = interaction + router_z + balance


@jax.jit
def _run(e_t, e_i, m_t, m_i, attn_W1, attn_b1, attn_W2, attn_b2,
         gate_W1, gate_b1, gate_W2, gate_b2):
    nblk = B // BLOCK
    row_spec = pl.BlockSpec((BLOCK, D), lambda i: (i, 0))
    full = lambda shape: pl.BlockSpec(shape, lambda i: (0,) * len(shape))

    mask, loss = pl.pallas_call(
        _fused_kernel,
        grid=(nblk,),
        in_specs=[
            row_spec, row_spec, row_spec, row_spec,
            full((D, D)), full((1, D)), full((D, 1)), full((1, 1)),
            full((D, D)), full((1, D)), full((D, 2)), full((1, 2)),
        ],
        out_specs=[
            pl.BlockSpec((BLOCK, 4), lambda i: (i, 0)),
            pl.BlockSpec(memory_space=pltpu.SMEM),
        ],
        out_shape=[
            jax.ShapeDtypeStruct((B, 4), jnp.float32),
            jax.ShapeDtypeStruct((1, 1), jnp.float32),
        ],
        scratch_shapes=[pltpu.SMEM((3,), jnp.float32)],
    )(e_t, e_i, m_t, m_i,
      attn_W1, attn_b1.reshape(1, D), attn_W2, attn_b2.reshape(1, 1),
      gate_W1, gate_b1.reshape(1, D), gate_W2, gate_b2.reshape(1, 2))
    return mask, loss[0, 0]


def kernel(p_t, p_i, e_t, e_i, m_t, m_i, attn_W1, attn_b1, attn_W2, attn_b2,
           gate_W1, gate_b1, gate_W2, gate_b2):
    return _run(e_t, e_i, m_t, m_i, attn_W1, attn_b1, attn_W2, attn_b2,
                gate_W1, gate_b1, gate_W2, gate_b2)


# speedup vs baseline: 1.1718x; 1.0365x over previous
import jax
import jax.numpy as jnp
from jax.experimental import pallas as pl
from jax.experimental.pallas import tpu as pltpu

B = 16384
D = 64
SEM_T = 0.3
IL_COEF = 0.7
BL_COEF = 0.1
RZ_COEF = 0.01

BLOCK = 4096
HALF = BLOCK // 2


def _silu(x):
    return x * (0.5 + 0.5 * jnp.tanh(0.5 * x))


def _dgen(a, b, ca, cb):
    return jax.lax.dot_general(a, b, (((ca,), (cb,)), ((), ())),
                               preferred_element_type=jnp.float32)


def _iota2(shape, dim):
    return jax.lax.broadcasted_iota(jnp.int32, shape, dim)


def _pair(v):
    return jnp.concatenate([v[:HALF, :], v[HALF:, :]], axis=1)


def _fused_kernel(et_ref, ei_ref, mt_ref, mi_ref,
                  aW1_ref, ab1_ref, aW2_ref, ab2_ref,
                  gW1_ref, gb1_ref, gW2_ref, gb2_ref,
                  mask_ref, loss_ref, acc_ref):
    i = pl.program_id(0)
    nblk = pl.num_programs(0)
    f32 = jnp.float32

    @pl.when(i == 0)
    def _init():
        acc_ref[...] = jnp.zeros((8, HALF), jnp.float32)

    x_et = _pair(et_ref[...])
    x_ei = _pair(ei_ref[...])
    x_mt = _pair(mt_ref[...])
    x_mi = _pair(mi_ref[...])

    dmask = (_iota2((2 * D, 2 * D), 0) // D) == (_iota2((2 * D, 2 * D), 1) // D)
    aW1d = jnp.where(dmask, jnp.tile(aW1_ref[...], (2, 2)), 0.0)
    gW1d = jnp.where(dmask, jnp.tile(gW1_ref[...], (2, 2)), 0.0)
    ab1d = jnp.tile(ab1_ref[...], (1, 2))
    gb1d = jnp.tile(gb1_ref[...], (1, 2))
    segsel = ((_iota2((2, 2 * D), 1) // D) == _iota2((2, 2 * D), 0)).astype(f32)
    w2pT = segsel * jnp.tile(jnp.transpose(aW2_ref[...]), (1, 2))
    gw2pT = jnp.where(
        (_iota2((4, 2 * D), 1) // D) == (_iota2((4, 2 * D), 0) // 2),
        jnp.tile(jnp.transpose(gW2_ref[...]), (2, 2)), 0.0)
    gb2T = jnp.transpose(jnp.tile(gb2_ref[...], (1, 2)))

    def score_t(xp):
        h = _silu(_dgen(xp, aW1d, 1, 0) + ab1d)
        return _dgen(w2pT, h, 1, 1) + ab2_ref[0, 0]

    s_et = score_t(x_et)
    s_ei = score_t(x_ei)
    s_mt = score_t(x_mt)
    s_mi = score_t(x_mi)

    gate_in = (x_et * _dgen(s_et, segsel, 0, 0)
               + x_ei * _dgen(s_ei, segsel, 0, 0)
               + x_mt * _dgen(s_mt, segsel, 0, 0)
               + x_mi * _dgen(s_mi, segsel, 0, 0))

    g = _silu(_dgen(gate_in, gW1d, 1, 0) + gb1d)
    logitsT = _dgen(gw2pT, g, 1, 1) + gb2T

    dotT = _dgen(segsel, x_mt * x_mi, 1, 1)
    ntT = _dgen(segsel, x_mt * x_mt, 1, 1)
    niT = _dgen(segsel, x_mi * x_mi, 1, 1)
    clip = dotT * jax.lax.rsqrt(ntT) * jax.lax.rsqrt(niT)
    sem1 = clip > SEM_T

    l0 = jnp.concatenate([logitsT[0:1, :], logitsT[2:3, :]], axis=0)
    l1 = jnp.concatenate([logitsT[1:2, :], logitsT[3:4, :]], axis=0)
    d = l0 - l1
    t = jnp.log1p(jnp.exp(-jnp.abs(d)))
    picked = -(t + jnp.where(sem1, jnp.maximum(d, 0.0), jnp.maximum(-d, 0.0)))
    lse = jnp.maximum(l0, l1) + t
    acc_ref[0:2, :] += picked
    acc_ref[2:4, :] += lse * lse
    acc_ref[4:6, :] += (l1 > l0).astype(f32)

    p0 = 0.5 + 0.5 * jnp.tanh(0.5 * d)
    p1 = 1.0 - p0
    eye4 = (_iota2((4, 4), 0) == _iota2((4, 4), 1)).astype(f32)

    def mask_rows(k):
        mT = jnp.concatenate([p0[k:k + 1], p0[k:k + 1],
                              p1[k:k + 1], p1[k:k + 1]], axis=0)
        return _dgen(mT, eye4, 0, 0)

    mask_ref[0:HALF, :] = mask_rows(0)
    mask_ref[HALF:BLOCK, :] = mask_rows(1)

    @pl.when(i == nblk - 1)
    def _final():
        inv_b = 1.0 / B
        interaction = IL_COEF * (-(jnp.sum(acc_ref[0:2, :]) * inv_b))
        router_z = RZ_COEF * (RZ_COEF * (jnp.sum(acc_ref[2:4, :]) * inv_b))
        d1 = jnp.sum(acc_ref[4:6, :]) * inv_b
        balance = BL_COEF * (d1 - 0.5) * (d1 - 0.5)
        loss_ref[0, 0] = interaction + router_z + balance


@jax.jit
def _run(e_t, e_i, m_t, m_i, attn_W1, attn_b1, attn_W2, attn_b2,
         gate_W1, gate_b1, gate_W2, gate_b2):
    nblk = B // BLOCK
    row_spec = pl.BlockSpec((BLOCK, D), lambda i: (i, 0))
    full = lambda shape: pl.BlockSpec(shape, lambda i: (0,) * len(shape))

    mask, loss = pl.pallas_call(
        _fused_kernel,
        grid=(nblk,),
        in_specs=[
            row_spec, row_spec, row_spec, row_spec,
            full((D, D)), full((1, D)), full((D, 1)), full((1, 1)),
            full((D, D)), full((1, D)), full((D, 2)), full((1, 2)),
        ],
        out_specs=[
            pl.BlockSpec((BLOCK, 4), lambda i: (i, 0)),
            pl.BlockSpec(memory_space=pltpu.SMEM),
        ],
        out_shape=[
            jax.ShapeDtypeStruct((B, 4), jnp.float32),
            jax.ShapeDtypeStruct((1, 1), jnp.float32),
        ],
        scratch_shapes=[pltpu.VMEM((8, HALF), jnp.float32)],
    )(e_t, e_i, m_t, m_i,
      attn_W1, attn_b1.reshape(1, D), attn_W2, attn_b2.reshape(1, 1),
      gate_W1, gate_b1.reshape(1, D), gate_W2, gate_b2.reshape(1, 2))
    return mask, loss[0, 0]


def kernel(p_t, p_i, e_t, e_i, m_t, m_i, attn_W1, attn_b1, attn_W2, attn_b2,
           gate_W1, gate_b1, gate_W2, gate_b2):
    return _run(e_t, e_i, m_t, m_i, attn_W1, attn_b1, attn_W2, attn_b2,
                gate_W1, gate_b1, gate_W2, gate_b2)


# speedup vs baseline: 1.1741x; 1.0020x over previous
import jax
import jax.numpy as jnp
from jax.experimental import pallas as pl
from jax.experimental.pallas import tpu as pltpu

B = 16384
D = 64
SEM_T = 0.3
IL_COEF = 0.7
BL_COEF = 0.1
RZ_COEF = 0.01

BLOCK = 4096
HALF = BLOCK // 2


def _silu(x):
    return x * (0.5 + 0.5 * jnp.tanh(0.5 * x))


def _dgen(a, b, ca, cb):
    return jax.lax.dot_general(a, b, (((ca,), (cb,)), ((), ())),
                               preferred_element_type=jnp.float32)


def _iota2(shape, dim):
    return jax.lax.broadcasted_iota(jnp.int32, shape, dim)


def _pair(v):
    return jnp.concatenate([v[:HALF, :], v[HALF:, :]], axis=1)


def _fused_kernel(et_ref, ei_ref, mt_ref, mi_ref,
                  aW1_ref, ab1_ref, aW2_ref, ab2_ref,
                  gW1_ref, gb1_ref, gW2_ref, gb2_ref,
                  mask_ref, loss_ref, acc_ref):
    i = pl.program_id(0)
    nblk = pl.num_programs(0)
    f32 = jnp.float32

    @pl.when(i == 0)
    def _init():
        acc_ref[0] = 0.0
        acc_ref[1] = 0.0
        acc_ref[2] = 0.0

    x_et = _pair(et_ref[...])
    x_ei = _pair(ei_ref[...])
    x_mt = _pair(mt_ref[...])
    x_mi = _pair(mi_ref[...])

    dmask = (_iota2((2 * D, 2 * D), 0) // D) == (_iota2((2 * D, 2 * D), 1) // D)
    aW1d = jnp.where(dmask, jnp.tile(aW1_ref[...], (2, 2)), 0.0)
    gW1d = jnp.where(dmask, jnp.tile(gW1_ref[...], (2, 2)), 0.0)
    ab1d = jnp.tile(ab1_ref[...], (1, 2))
    gb1d = jnp.tile(gb1_ref[...], (1, 2))
    segsel = ((_iota2((2, 2 * D), 1) // D) == _iota2((2, 2 * D), 0)).astype(f32)
    w2pT = segsel * jnp.tile(jnp.transpose(aW2_ref[...]), (1, 2))
    gw2pT = jnp.where(
        (_iota2((4, 2 * D), 1) // D) == (_iota2((4, 2 * D), 0) // 2),
        jnp.tile(jnp.transpose(gW2_ref[...]), (2, 2)), 0.0)
    gb2T = jnp.transpose(jnp.tile(gb2_ref[...], (1, 2)))

    def score_t(xp):
        h = _silu(_dgen(xp, aW1d, 1, 0) + ab1d)
        return _dgen(w2pT, h, 1, 1) + ab2_ref[0, 0]

    s_et = score_t(x_et)
    s_ei = score_t(x_ei)
    s_mt = score_t(x_mt)
    s_mi = score_t(x_mi)

    gate_in = (x_et * _dgen(s_et, segsel, 0, 0)
               + x_ei * _dgen(s_ei, segsel, 0, 0)
               + x_mt * _dgen(s_mt, segsel, 0, 0)
               + x_mi * _dgen(s_mi, segsel, 0, 0))

    g = _silu(_dgen(gate_in, gW1d, 1, 0) + gb1d)
    logitsT = _dgen(gw2pT, g, 1, 1) + gb2T

    dotT = _dgen(segsel, x_mt * x_mi, 1, 1)
    ntT = _dgen(segsel, x_mt * x_mt, 1, 1)
    niT = _dgen(segsel, x_mi * x_mi, 1, 1)
    clip = dotT * jax.lax.rsqrt(ntT) * jax.lax.rsqrt(niT)
    sem1 = clip > SEM_T

    l0 = jnp.concatenate([logitsT[0:1, :], logitsT[2:3, :]], axis=0)
    l1 = jnp.concatenate([logitsT[1:2, :], logitsT[3:4, :]], axis=0)
    d = l0 - l1
    t = jnp.log1p(jnp.exp(-jnp.abs(d)))
    picked = -(t + jnp.where(sem1, jnp.maximum(d, 0.0), jnp.maximum(-d, 0.0)))
    lse = jnp.maximum(l0, l1) + t
    acc_ref[0] += jnp.sum(picked)
    acc_ref[1] += jnp.sum(lse * lse)
    acc_ref[2] += jnp.sum((l1 > l0).astype(f32))

    p0 = 0.5 + 0.5 * jnp.tanh(0.5 * d)
    p1 = 1.0 - p0
    eye4 = (_iota2((4, 4), 0) == _iota2((4, 4), 1)).astype(f32)

    def mask_rows(k):
        mT = jnp.concatenate([p0[k:k + 1], p0[k:k + 1],
                              p1[k:k + 1], p1[k:k + 1]], axis=0)
        return _dgen(mT, eye4, 0, 0)

    mask_ref[0:HALF, :] = mask_rows(0)
    mask_ref[HALF:BLOCK, :] = mask_rows(1)

    @pl.when(i == nblk - 1)
    def _final():
        inv_b = 1.0 / B
        interaction = IL_COEF * (-(acc_ref[0] * inv_b))
        router_z = RZ_COEF * (RZ_COEF * (acc_ref[1] * inv_b))
        d1 = acc_ref[2] * inv_b
        balance = BL_COEF * (d1 - 0.5) * (d1 - 0.5)
        loss_ref[0, 0] = interaction + router_z + balance


@jax.jit
def _run(e_t, e_i, m_t, m_i, attn_W1, attn_b1, attn_W2, attn_b2,
         gate_W1, gate_b1, gate_W2, gate_b2):
    nblk = B // BLOCK
    row_spec = pl.BlockSpec((BLOCK, D), lambda i: (i, 0))
    full = lambda shape: pl.BlockSpec(shape, lambda i: (0,) * len(shape))

    mask, loss = pl.pallas_call(
        _fused_kernel,
        grid=(nblk,),
        in_specs=[
            row_spec, row_spec, row_spec, row_spec,
            full((D, D)), full((1, D)), full((D, 1)), full((1, 1)),
            full((D, D)), full((1, D)), full((D, 2)), full((1, 2)),
        ],
        out_specs=[
            pl.BlockSpec((BLOCK, 4), lambda i: (i, 0)),
            pl.BlockSpec(memory_space=pltpu.SMEM),
        ],
        out_shape=[
            jax.ShapeDtypeStruct((B, 4), jnp.float32),
            jax.ShapeDtypeStruct((1, 1), jnp.float32),
        ],
        scratch_shapes=[pltpu.SMEM((3,), jnp.float32)],
    )(e_t, e_i, m_t, m_i,
      attn_W1, attn_b1.reshape(1, D), attn_W2, attn_b2.reshape(1, 1),
      gate_W1, gate_b1.reshape(1, D), gate_W2, gate_b2.reshape(1, 2))
    return mask, loss[0, 0]


def kernel(p_t, p_i, e_t, e_i, m_t, m_i, attn_W1, attn_b1, attn_W2, attn_b2,
           gate_W1, gate_b1, gate_W2, gate_b2):
    return _run(e_t, e_i, m_t, m_i, attn_W1, attn_b1, attn_W2, attn_b2,
                gate_W1, gate_b1, gate_W2, gate_b2)
